# baseline (speedup 1.0000x reference)
import jax
import jax.numpy as jnp
from jax import lax
from jax.experimental import pallas as pl
from jax.experimental.pallas import tpu as pltpu

_R = 256
_SH = 8


def _med3(a, b, c):
    mn = jnp.minimum(a, b)
    mx = jnp.maximum(a, b)
    return jnp.maximum(mn, jnp.minimum(mx, c))


def _strip(cur, pv, nv, w):
    up = jnp.concatenate([pv, cur[: _SH - 1, :]], axis=0)
    dn = jnp.concatenate([cur[1:, :], nv], axis=0)

    mn = jnp.minimum(up, dn)
    mx = jnp.maximum(up, dn)
    lo = jnp.minimum(mn, cur)
    hi = jnp.maximum(mx, cur)
    mi = jnp.maximum(mn, jnp.minimum(mx, cur))

    def hshifts(t):
        lt = jnp.concatenate([t[:, 1:2], t[:, : w - 1]], axis=1)
        rt = jnp.concatenate([t[:, 1:], t[:, w - 2 : w - 1]], axis=1)
        return lt, rt

    lo_l, lo_r = hshifts(lo)
    hi_l, hi_r = hshifts(hi)
    mi_l, mi_r = hshifts(mi)

    a = jnp.maximum(jnp.maximum(lo_l, lo), lo_r)
    c = jnp.minimum(jnp.minimum(hi_l, hi), hi_r)
    b = _med3(mi_l, mi, mi_r)
    return _med3(a, b, c)


def _median3x3_kernel(top_ref, x_ref, bot_ref, o_ref):
    rows = x_ref.shape[1]
    w = x_ref.shape[2]
    n = rows // _SH
    h = pl.program_id(1)
    nh = pl.num_programs(1)

    pv_top = jnp.where(h == 0, top_ref[0, 1:2, :], top_ref[0, 7:8, :])
    nv_bot = jnp.where(h == nh - 1, bot_ref[0, 6:7, :], bot_ref[0, 0:1, :])

    cur = x_ref[0, 0:_SH, :]
    nv = x_ref[0, _SH : _SH + 1, :]
    o_ref[0, 0:_SH, :] = _strip(cur, pv_top, nv, w)

    def body(s, _):
        r0 = pl.multiple_of(s * _SH, _SH)
        cur = x_ref[0, pl.ds(r0, _SH), :]
        pv = x_ref[0, pl.ds(r0 - 1, 1), :]
        nv = x_ref[0, pl.ds(r0 + _SH, 1), :]
        o_ref[0, pl.ds(r0, _SH), :] = _strip(cur, pv, nv, w)
        return 0

    lax.fori_loop(1, n - 1, body, 0, unroll=5)

    r0 = (n - 1) * _SH
    cur = x_ref[0, r0 : r0 + _SH, :]
    pv = x_ref[0, r0 - 1 : r0, :]
    o_ref[0, r0 : r0 + _SH, :] = _strip(cur, pv, nv_bot, w)


def kernel(img):
    B, C, H, W = img.shape
    r = _R if H % _R == 0 else H
    rb = r // 8
    x = img.reshape(B * C, H, W)
    hb = H // 8

    out = pl.pallas_call(
        _median3x3_kernel,
        out_shape=jax.ShapeDtypeStruct((B * C, H, W), img.dtype),
        grid=(B * C, H // r),
        in_specs=[
            pl.BlockSpec(
                (1, 8, W), lambda i, h: (i, jnp.maximum(h * rb - 1, 0), 0)
            ),
            pl.BlockSpec((1, r, W), lambda i, h: (i, h, 0)),
            pl.BlockSpec(
                (1, 8, W),
                lambda i, h: (i, jnp.minimum(h * rb + rb, hb - 1), 0),
            ),
        ],
        out_specs=pl.BlockSpec((1, r, W), lambda i, h: (i, h, 0)),
        compiler_params=pltpu.CompilerParams(
            dimension_semantics=("parallel", "arbitrary"),
        ),
        name="median3x3",
    )(x, x, x)
    return out.reshape(B, C, H, W)


# speedup vs baseline: 1.3825x; 1.3825x over previous
import jax
import jax.numpy as jnp
from jax.experimental import pallas as pl
from jax.experimental.pallas import tpu as pltpu

_R = 1024


def _med3(a, b, c):
    mn = jnp.minimum(a, b)
    mx = jnp.maximum(a, b)
    return jnp.maximum(mn, jnp.minimum(mx, c))


def _median3x3_kernel(top_ref, x_ref, bot_ref, o_ref):
    rows = x_ref.shape[1]
    w = x_ref.shape[2]
    sh = 8
    n = rows // sh
    h = pl.program_id(1)
    nh = pl.num_programs(1)

    pv_top = jnp.where(h == 0, top_ref[0, 1:2, :], top_ref[0, 7:8, :])
    nv_bot = jnp.where(h == nh - 1, bot_ref[0, 6:7, :], bot_ref[0, 0:1, :])

    def hshifts(t):
        lt = jnp.concatenate([t[:, 1:2], t[:, : w - 1]], axis=1)
        rt = jnp.concatenate([t[:, 1:], t[:, w - 2 : w - 1]], axis=1)
        return lt, rt

    def vstage(s):
        cur = x_ref[0, s * sh : (s + 1) * sh, :]
        pv = pv_top if s == 0 else x_ref[0, s * sh - 1 : s * sh, :]
        nv = nv_bot if s == n - 1 else x_ref[0, (s + 1) * sh : (s + 1) * sh + 1, :]

        up = jnp.concatenate([pv, cur[: sh - 1, :]], axis=0)
        dn = jnp.concatenate([cur[1:, :], nv], axis=0)

        mn = jnp.minimum(up, dn)
        mx = jnp.maximum(up, dn)
        lo = jnp.minimum(mn, cur)
        hi = jnp.maximum(mx, cur)
        mi = jnp.maximum(mn, jnp.minimum(mx, cur))
        return (lo, mi, hi, hshifts(lo), hshifts(mi), hshifts(hi))

    def combine(s, st):
        lo, mi, hi, (lo_l, lo_r), (mi_l, mi_r), (hi_l, hi_r) = st
        a = jnp.maximum(jnp.maximum(lo_l, lo), lo_r)
        c = jnp.minimum(jnp.minimum(hi_l, hi), hi_r)
        b = _med3(mi_l, mi, mi_r)
        o_ref[0, s * sh : (s + 1) * sh, :] = _med3(a, b, c)

    st = vstage(0)
    for s in range(n):
        nxt = vstage(s + 1) if s + 1 < n else None
        combine(s, st)
        st = nxt


def kernel(img):
    B, C, H, W = img.shape
    r = _R if H % _R == 0 else H
    rb = r // 8
    x = img.reshape(B * C, H, W)
    hb = H // 8

    out = pl.pallas_call(
        _median3x3_kernel,
        out_shape=jax.ShapeDtypeStruct((B * C, H, W), img.dtype),
        grid=(B * C, H // r),
        in_specs=[
            pl.BlockSpec(
                (1, 8, W), lambda i, h: (i, jnp.maximum(h * rb - 1, 0), 0)
            ),
            pl.BlockSpec((1, r, W), lambda i, h: (i, h, 0)),
            pl.BlockSpec(
                (1, 8, W),
                lambda i, h: (i, jnp.minimum(h * rb + rb, hb - 1), 0),
            ),
        ],
        out_specs=pl.BlockSpec((1, r, W), lambda i, h: (i, h, 0)),
        compiler_params=pltpu.CompilerParams(
            dimension_semantics=("parallel", "arbitrary"),
        ),
        name="median3x3",
    )(x, x, x)
    return out.reshape(B, C, H, W)


# speedup vs baseline: 1.6160x; 1.1689x over previous
import jax
import jax.numpy as jnp
from jax.experimental import pallas as pl
from jax.experimental.pallas import tpu as pltpu

_R = 1024
_SH = 8


def _med3(a, b, c):
    mn = jnp.minimum(a, b)
    mx = jnp.maximum(a, b)
    return jnp.maximum(mn, jnp.minimum(mx, c))


def _median3x3_kernel(top_ref, x_ref, bot_ref, o_ref):
    rows = x_ref.shape[1]
    w = x_ref.shape[2]
    n = rows // _SH
    h = pl.program_id(1)
    nh = pl.num_programs(1)

    pv_top = jnp.where(h == 0, top_ref[0, 1:2, :], top_ref[0, 7:8, :])
    nv_bot = jnp.where(h == nh - 1, bot_ref[0, 6:7, :], bot_ref[0, 0:1, :])

    def hsort(t):
        tl = jnp.concatenate([t[:, 1:2], t[:, : w - 1]], axis=1)
        tr = jnp.concatenate([t[:, 1:], t[:, w - 2 : w - 1]], axis=1)
        mn = jnp.minimum(tl, tr)
        mx = jnp.maximum(tl, tr)
        lo = jnp.minimum(mn, t)
        hi = jnp.maximum(mx, t)
        mi = jnp.maximum(mn, jnp.minimum(mx, t))
        return lo, mi, hi

    def hstage(s):
        return hsort(x_ref[0, s * _SH : (s + 1) * _SH, :])

    def vcombine(s, prev3, cur3, nxt3):
        lo, mi, hi = cur3
        plo, pmi, phi = prev3
        nlo, nmi, nhi = nxt3

        def vshifts(t, pt, nt):
            up = jnp.concatenate([pt, t[: _SH - 1, :]], axis=0)
            dn = jnp.concatenate([t[1:, :], nt], axis=0)
            return up, dn

        lo_u, lo_d = vshifts(lo, plo, nlo)
        mi_u, mi_d = vshifts(mi, pmi, nmi)
        hi_u, hi_d = vshifts(hi, phi, nhi)

        a = jnp.maximum(jnp.maximum(lo_u, lo), lo_d)
        c = jnp.minimum(jnp.minimum(hi_u, hi), hi_d)
        b = _med3(mi_u, mi, mi_d)
        o_ref[0, s * _SH : (s + 1) * _SH, :] = _med3(a, b, c)

    th = hsort(pv_top)
    bh = hsort(nv_bot)

    cur3 = hstage(0)
    nxt3 = hstage(1) if n > 1 else None
    for s in range(n):
        prev_edge = th if s == 0 else tuple(t[_SH - 1 : _SH, :] for t in prev3)
        next_edge = bh if s == n - 1 else tuple(t[0:1, :] for t in nxt3)
        nxt2 = hstage(s + 2) if s + 2 < n else None
        vcombine(s, prev_edge, cur3, next_edge)
        prev3, cur3, nxt3 = cur3, nxt3, nxt2


def kernel(img):
    B, C, H, W = img.shape
    r = _R if H % _R == 0 else H
    rb = r // 8
    x = img.reshape(B * C, H, W)
    hb = H // 8

    out = pl.pallas_call(
        _median3x3_kernel,
        out_shape=jax.ShapeDtypeStruct((B * C, H, W), img.dtype),
        grid=(B * C, H // r),
        in_specs=[
            pl.BlockSpec(
                (1, 8, W), lambda i, h: (i, jnp.maximum(h * rb - 1, 0), 0)
            ),
            pl.BlockSpec((1, r, W), lambda i, h: (i, h, 0)),
            pl.BlockSpec(
                (1, 8, W),
                lambda i, h: (i, jnp.minimum(h * rb + rb, hb - 1), 0),
            ),
        ],
        out_specs=pl.BlockSpec((1, r, W), lambda i, h: (i, h, 0)),
        compiler_params=pltpu.CompilerParams(
            dimension_semantics=("parallel", "arbitrary"),
        ),
        name="median3x3",
    )(x, x, x)
    return out.reshape(B, C, H, W)
